# Initial kernel scaffold; baseline (speedup 1.0000x reference)
import jax
import jax.numpy as jnp
from jax.experimental import pallas as pl


def kernel(candidate_news_topicindex, table, W1, b1, W2, b2):
    raise NotImplementedError("write your pallas kernel here")


# speedup vs baseline: 2.2615x; 2.2615x over previous
import functools

import jax
import jax.numpy as jnp
from jax import lax
from jax.experimental import pallas as pl
from jax.experimental.pallas import tpu as pltpu
from jax.experimental.pallas import tpu_sc as plsc

NUM_WORKERS = 32
CHUNK = 128


def _mlp_table_body(tab_ref, w1_ref, b1_ref, w2_ref, b2_ref, out_ref):
    x = tab_ref[...]
    h = lax.dot_general(x, w1_ref[...], (((1,), (1,)), ((), ())),
                        preferred_element_type=jnp.float32)
    h = jnp.maximum(h + b1_ref[...], 0.0)
    o = lax.dot_general(h, w2_ref[...], (((1,), (1,)), ((), ())),
                        preferred_element_type=jnp.float32)
    out_ref[...] = o + b2_ref[...]


def _transform_table(table, W1, b1, W2, b2):
    n, d = table.shape
    return pl.pallas_call(
        _mlp_table_body,
        out_shape=jax.ShapeDtypeStruct((n, d), jnp.float32),
    )(table, W1, b1.reshape(1, d), W2, b2.reshape(1, d))


def _make_gather(B, D):
    b_per_w = B // NUM_WORKERS
    n_chunks = b_per_w // CHUNK
    mesh = plsc.VectorSubcoreMesh(core_axis_name="c", subcore_axis_name="s")

    @functools.partial(
        pl.kernel,
        mesh=mesh,
        out_type=jax.ShapeDtypeStruct((B, D), jnp.float32),
        scratch_types=[
            pltpu.VMEM((CHUNK,), jnp.int32),
            pltpu.VMEM((CHUNK, D), jnp.float32),
            pltpu.SemaphoreType.DMA,
        ],
    )
    def gather_kernel(tab_hbm, idx_hbm, out_hbm, idx_v, rows_v, sem):
        wid = lax.axis_index("s") * 2 + lax.axis_index("c")
        base = wid * b_per_w

        def body(c, carry):
            off = base + c * CHUNK
            pltpu.sync_copy(idx_hbm.at[pl.ds(off, CHUNK)], idx_v)
            pltpu.async_copy(tab_hbm.at[idx_v], rows_v, sem).wait()
            pltpu.sync_copy(rows_v, out_hbm.at[pl.ds(off, CHUNK)])
            return carry

        lax.fori_loop(0, n_chunks, body, 0)

    return gather_kernel


def kernel(candidate_news_topicindex, table, W1, b1, W2, b2):
    bsz, seq = candidate_news_topicindex.shape
    d = table.shape[1]
    idx_flat = candidate_news_topicindex.astype(jnp.int32).reshape(-1)
    tab_t = _transform_table(table, W1, b1, W2, b2)
    out = _make_gather(idx_flat.shape[0], d)(tab_t, idx_flat)
    return out.reshape(bsz, seq, d)


# speedup vs baseline: 2.2809x; 1.0086x over previous
import functools

import jax
import jax.numpy as jnp
from jax import lax
from jax.experimental import pallas as pl
from jax.experimental.pallas import tpu as pltpu
from jax.experimental.pallas import tpu_sc as plsc

NUM_WORKERS = 32
CHUNK = 128
NBUF = 4


def _mlp_table_body(tab_ref, w1_ref, b1_ref, w2_ref, b2_ref, out_ref):
    x = tab_ref[...]
    h = lax.dot_general(x, w1_ref[...], (((1,), (1,)), ((), ())),
                        preferred_element_type=jnp.float32)
    h = jnp.maximum(h + b1_ref[...], 0.0)
    o = lax.dot_general(h, w2_ref[...], (((1,), (1,)), ((), ())),
                        preferred_element_type=jnp.float32)
    out_ref[...] = o + b2_ref[...]


def _transform_table(table, W1, b1, W2, b2):
    n, d = table.shape
    return pl.pallas_call(
        _mlp_table_body,
        out_shape=jax.ShapeDtypeStruct((n, d), jnp.float32),
    )(table, W1, b1.reshape(1, d), W2, b2.reshape(1, d))


def _make_gather(B, D):
    b_per_w = B // NUM_WORKERS
    n_chunks = b_per_w // CHUNK
    n_groups = n_chunks // NBUF
    mesh = plsc.VectorSubcoreMesh(core_axis_name="c", subcore_axis_name="s")

    @functools.partial(
        pl.kernel,
        mesh=mesh,
        out_type=jax.ShapeDtypeStruct((B, D), jnp.float32),
        scratch_types=[
            pltpu.VMEM((b_per_w,), jnp.int32),
            pltpu.VMEM((NBUF, CHUNK, D), jnp.float32),
            [pltpu.SemaphoreType.DMA] * NBUF,
            [pltpu.SemaphoreType.DMA] * NBUF,
        ],
    )
    def gather_kernel(tab_hbm, idx_hbm, out_hbm, idx_v, rows, gsems, ssems):
        wid = lax.axis_index("s") * 2 + lax.axis_index("c")
        base = wid * b_per_w
        pltpu.sync_copy(idx_hbm.at[pl.ds(base, b_per_w)], idx_v)

        def start_gather(c, j):
            pltpu.async_copy(
                tab_hbm.at[idx_v.at[pl.ds(c * CHUNK, CHUNK)]], rows.at[j],
                gsems[j])

        def wait_gather(j):
            pltpu.make_async_copy(
                tab_hbm.at[idx_v.at[pl.ds(0, CHUNK)]], rows.at[j],
                gsems[j]).wait()

        def start_store(c, j):
            pltpu.async_copy(
                rows.at[j], out_hbm.at[pl.ds(base + c * CHUNK, CHUNK)],
                ssems[j])

        def wait_store(j):
            pltpu.make_async_copy(
                rows.at[j], out_hbm.at[pl.ds(base, CHUNK)], ssems[j]).wait()

        for j in range(NBUF):
            start_gather(j, j)

        def body(g, carry):
            c0 = g * NBUF
            for j in range(NBUF):
                wait_gather(j)
                start_store(c0 + j, j)
            for j in range(NBUF):
                wait_store(j)
                start_gather(c0 + NBUF + j, j)
            return carry

        lax.fori_loop(0, n_groups - 1, body, 0)

        c0 = (n_groups - 1) * NBUF
        for j in range(NBUF):
            wait_gather(j)
            start_store(c0 + j, j)
        for j in range(NBUF):
            wait_store(j)

    return gather_kernel


def kernel(candidate_news_topicindex, table, W1, b1, W2, b2):
    bsz, seq = candidate_news_topicindex.shape
    d = table.shape[1]
    idx_flat = candidate_news_topicindex.astype(jnp.int32).reshape(-1)
    tab_t = _transform_table(table, W1, b1, W2, b2)
    out = _make_gather(idx_flat.shape[0], d)(tab_t, idx_flat)
    return out.reshape(bsz, seq, d)


# speedup vs baseline: 3.6346x; 1.5935x over previous
import functools

import jax
import jax.numpy as jnp
from jax import lax
from jax.experimental import pallas as pl
from jax.experimental.pallas import tpu as pltpu
from jax.experimental.pallas import tpu_sc as plsc

NUM_WORKERS = 32
CHUNK = 128
NBUF = 4


def _mlp_table_body(tab_ref, w1_ref, b1_ref, w2_ref, b2_ref, out_ref):
    x = tab_ref[...]
    h = lax.dot_general(x, w1_ref[...], (((1,), (1,)), ((), ())),
                        preferred_element_type=jnp.float32)
    h = jnp.maximum(h + b1_ref[...], 0.0)
    o = lax.dot_general(h, w2_ref[...], (((1,), (1,)), ((), ())),
                        preferred_element_type=jnp.float32)
    out_ref[...] = o + b2_ref[...]


def _transform_table(table, W1, b1, W2, b2):
    n, d = table.shape
    return pl.pallas_call(
        _mlp_table_body,
        out_shape=jax.ShapeDtypeStruct((n, d), jnp.float32),
    )(table, W1, b1.reshape(1, d), W2, b2.reshape(1, d))


def _make_gather(B, D, N):
    b_per_w = B // NUM_WORKERS
    n_chunks = b_per_w // CHUNK
    n_groups = n_chunks // NBUF
    mesh = plsc.VectorSubcoreMesh(core_axis_name="c", subcore_axis_name="s")

    @functools.partial(
        pl.kernel,
        mesh=mesh,
        out_type=jax.ShapeDtypeStruct((B, D), jnp.float32),
        scratch_types=[
            pltpu.VMEM((b_per_w,), jnp.int32),
            pltpu.VMEM((NBUF, CHUNK, D), jnp.float32),
            pltpu.VMEM_SHARED((N, D), jnp.float32),
            [pltpu.SemaphoreType.DMA] * NBUF,
            [pltpu.SemaphoreType.DMA] * NBUF,
        ],
    )
    def gather_kernel(tab_hbm, idx_hbm, out_hbm, idx_v, rows, tab_sh,
                      gsems, ssems):
        wid = lax.axis_index("s") * 2 + lax.axis_index("c")
        base = wid * b_per_w

        @pl.when(lax.axis_index("s") == 0)
        def _stage():
            pltpu.sync_copy(tab_hbm, tab_sh)

        pltpu.sync_copy(idx_hbm.at[pl.ds(base, b_per_w)], idx_v)
        plsc.subcore_barrier()

        def start_gather(c, j):
            pltpu.async_copy(
                tab_sh.at[idx_v.at[pl.ds(c * CHUNK, CHUNK)]], rows.at[j],
                gsems[j])

        def wait_gather(j):
            pltpu.make_async_copy(
                tab_sh.at[idx_v.at[pl.ds(0, CHUNK)]], rows.at[j],
                gsems[j]).wait()

        def start_store(c, j):
            pltpu.async_copy(
                rows.at[j], out_hbm.at[pl.ds(base + c * CHUNK, CHUNK)],
                ssems[j])

        def wait_store(j):
            pltpu.make_async_copy(
                rows.at[j], out_hbm.at[pl.ds(base, CHUNK)], ssems[j]).wait()

        for j in range(NBUF):
            start_gather(j, j)

        def body(g, carry):
            c0 = g * NBUF
            for j in range(NBUF):
                wait_gather(j)
                start_store(c0 + j, j)
            for j in range(NBUF):
                wait_store(j)
                start_gather(c0 + NBUF + j, j)
            return carry

        lax.fori_loop(0, n_groups - 1, body, 0)

        c0 = (n_groups - 1) * NBUF
        for j in range(NBUF):
            wait_gather(j)
            start_store(c0 + j, j)
        for j in range(NBUF):
            wait_store(j)

    return gather_kernel


def kernel(candidate_news_topicindex, table, W1, b1, W2, b2):
    bsz, seq = candidate_news_topicindex.shape
    d = table.shape[1]
    idx_flat = candidate_news_topicindex.astype(jnp.int32).reshape(-1)
    tab_t = _transform_table(table, W1, b1, W2, b2)
    out = _make_gather(idx_flat.shape[0], d, tab_t.shape[0])(tab_t, idx_flat)
    return out.reshape(bsz, seq, d)


# speedup vs baseline: 6.9747x; 1.9190x over previous
import functools

import jax
import jax.numpy as jnp
from jax import lax
from jax.experimental import pallas as pl
from jax.experimental.pallas import tpu as pltpu
from jax.experimental.pallas import tpu_sc as plsc

NUM_WORKERS = 32
NBUF = 4
SUBL = 8


def _mlp_table_body(tab_ref, w1_ref, b1_ref, w2_ref, b2_ref, out_ref):
    x = tab_ref[...]
    h = lax.dot_general(x, w1_ref[...], (((1,), (1,)), ((), ())),
                        preferred_element_type=jnp.float32)
    h = jnp.maximum(h + b1_ref[...], 0.0)
    o = lax.dot_general(h, w2_ref[...], (((1,), (1,)), ((), ())),
                        preferred_element_type=jnp.float32)
    out_ref[...] = o + b2_ref[...]


def _transform_table(table, W1, b1, W2, b2):
    n, d = table.shape
    return pl.pallas_call(
        _mlp_table_body,
        out_shape=jax.ShapeDtypeStruct((n, d), jnp.float32),
    )(table, W1, b1.reshape(1, d), W2, b2.reshape(1, d))


def _make_gather(BI, S, D, N):
    spad = (S + SUBL - 1) // SUBL * SUBL
    i_per_w = BI // NUM_WORKERS
    chunk = 2 * spad
    n_chunks = i_per_w // 2
    n_groups = n_chunks // NBUF
    mesh = plsc.VectorSubcoreMesh(core_axis_name="c", subcore_axis_name="s")

    @functools.partial(
        pl.kernel,
        mesh=mesh,
        out_type=jax.ShapeDtypeStruct((BI, S, D), jnp.float32),
        scratch_types=[
            pltpu.VMEM((i_per_w * spad,), jnp.int32),
            pltpu.VMEM((NBUF, chunk, D), jnp.float32),
            pltpu.VMEM_SHARED((N, D), jnp.float32),
            [pltpu.SemaphoreType.DMA] * NBUF,
            [pltpu.SemaphoreType.DMA] * NBUF,
        ],
        compiler_params=pltpu.CompilerParams(use_tc_tiling_on_sc=True),
    )
    def gather_kernel(tab_hbm, idxp_hbm, out_hbm, idx_v, rows, tab_sh,
                      gsems, ssems):
        wid = lax.axis_index("s") * 2 + lax.axis_index("c")
        ibase = wid * i_per_w

        @pl.when(lax.axis_index("s") == 0)
        def _stage():
            pltpu.sync_copy(tab_hbm, tab_sh)

        pltpu.sync_copy(
            idxp_hbm.at[pl.ds(ibase * spad, i_per_w * spad)], idx_v)
        plsc.subcore_barrier()

        def start_gather(c, j):
            pltpu.async_copy(
                tab_sh.at[idx_v.at[pl.ds(c * chunk, chunk)]], rows.at[j],
                gsems[j])

        def wait_gather(j):
            pltpu.make_async_copy(
                tab_sh.at[idx_v.at[pl.ds(0, chunk)]], rows.at[j],
                gsems[j]).wait()

        def start_store(c, j):
            i0 = ibase + 2 * c
            pltpu.async_copy(
                rows.at[j].at[pl.ds(0, S)], out_hbm.at[i0], ssems[j])
            pltpu.async_copy(
                rows.at[j].at[pl.ds(spad, S)], out_hbm.at[i0 + 1], ssems[j])

        def wait_store(j):
            pltpu.make_async_copy(
                rows.at[j].at[pl.ds(0, S)], out_hbm.at[ibase], ssems[j]).wait()
            pltpu.make_async_copy(
                rows.at[j].at[pl.ds(spad, S)], out_hbm.at[ibase],
                ssems[j]).wait()

        for j in range(NBUF):
            start_gather(j, j)

        def body(g, carry):
            c0 = g * NBUF
            for j in range(NBUF):
                wait_gather(j)
                start_store(c0 + j, j)
            for j in range(NBUF):
                wait_store(j)
                start_gather(c0 + NBUF + j, j)
            return carry

        lax.fori_loop(0, n_groups - 1, body, 0)

        c0 = (n_groups - 1) * NBUF
        for j in range(NBUF):
            wait_gather(j)
            start_store(c0 + j, j)
        for j in range(NBUF):
            wait_store(j)

    return gather_kernel


def kernel(candidate_news_topicindex, table, W1, b1, W2, b2):
    bsz, seq = candidate_news_topicindex.shape
    n, d = table.shape
    spad = (seq + SUBL - 1) // SUBL * SUBL
    idx32 = candidate_news_topicindex.astype(jnp.int32)
    idx_pad = jnp.pad(idx32, ((0, 0), (0, spad - seq))).reshape(-1)
    tab_t = _transform_table(table, W1, b1, W2, b2)
    return _make_gather(bsz, seq, d, n)(tab_t, idx_pad)
